# Initial kernel scaffold; baseline (speedup 1.0000x reference)
import jax
import jax.numpy as jnp
from jax.experimental import pallas as pl


def kernel(p, x, o, W1, b1, gamma, beta, W2, b2):
    raise NotImplementedError("write your pallas kernel here")


# speedup vs baseline: 4.6641x; 4.6641x over previous
import jax
import jax.numpy as jnp
from jax import lax
from jax.experimental import pallas as pl

_EPS = 1e-5


def _body(o_ref, x_ref, W1_ref, b1_ref, gamma_ref, beta_ref, W2_ref, b2_ref,
          out_ref):
    n, d = x_ref.shape
    nb = o_ref.shape[1]
    x = x_ref[...]
    o = o_ref[...]
    prev = jnp.concatenate(
        [jnp.zeros((1, 1), jnp.int32), o[:, :-1]], axis=1)
    r = lax.broadcasted_iota(jnp.int32, (n, nb), 0)
    maskf = ((r < o) & (r >= prev)).astype(jnp.float32)
    cnt = (o - prev).astype(jnp.float32)
    mask_mean = maskf * (1.0 / cnt)
    m = lax.dot_general(mask_mean, x, (((0,), (0,)), ((), ())),
                        preferred_element_type=jnp.float32)
    h = jnp.maximum(
        lax.dot_general(m, W2_ref[...], (((1,), (1,)), ((), ())),
                        preferred_element_type=jnp.float32) + b2_ref[...],
        0.0)
    W1 = W1_ref[...]
    W1a = W1[:, :d]
    W1b = W1[:, d:]
    c = lax.dot_general(h, W1b, (((1,), (1,)), ((), ())),
                        preferred_element_type=jnp.float32) + b1_ref[...]
    t = lax.dot_general(x, W1a, (((1,), (1,)), ((), ())),
                        preferred_element_type=jnp.float32)
    y = t + lax.dot_general(maskf, c, (((1,), (0,)), ((), ())),
                            preferred_element_type=jnp.float32)
    mean = jnp.mean(y, axis=0, keepdims=True)
    dev = y - mean
    var = jnp.mean(dev * dev, axis=0, keepdims=True)
    out = dev * lax.rsqrt(var + _EPS) * gamma_ref[...] + beta_ref[...]
    out_ref[...] = jnp.maximum(out, 0.0)


def kernel(p, x, o, W1, b1, gamma, beta, W2, b2):
    del p
    n, d = x.shape
    nb = o.shape[0]
    o2 = o.reshape(1, nb)
    return pl.pallas_call(
        _body,
        out_shape=jax.ShapeDtypeStruct((n, d), x.dtype),
    )(o2, x, W1, b1.reshape(1, d), gamma.reshape(1, d), beta.reshape(1, d),
      W2, b2.reshape(1, d))


# speedup vs baseline: 6.1065x; 1.3093x over previous
import jax
import jax.numpy as jnp
from jax import lax
from jax.experimental import pallas as pl

_EPS = 1e-5


def _dot(a, b, dims):
    return lax.dot_general(a, b, (dims, ((), ())),
                           preferred_element_type=jnp.float32)


def _body(o_ref, x_ref, W1_ref, b1_ref, gamma_ref, beta_ref, W2_ref, b2_ref,
          out_ref):
    n, d = x_ref.shape
    nb = o_ref.shape[0]
    nf = jnp.float32(n)
    x = x_ref[...]
    o = o_ref[...]
    o_prev = jnp.concatenate(
        [jnp.zeros((1, 1), jnp.int32), o[:-1, :]], axis=0)
    r = lax.broadcasted_iota(jnp.int32, (nb, n), 1)
    ltf = (r < o).astype(jnp.float32)
    lt_prev = jnp.concatenate(
        [jnp.zeros((1, n), jnp.float32), ltf[:-1, :]], axis=0)
    maskT = ltf - lt_prev
    cnt = (o - o_prev).astype(jnp.float32)

    sums = _dot(maskT, x, ((1,), (0,)))
    m = sums * (1.0 / cnt)
    h = jnp.maximum(_dot(m, W2_ref[...], ((1,), (1,))) + b2_ref[...], 0.0)
    W1 = W1_ref[...]
    W1a = W1[:, :d]
    W1b = W1[:, d:]
    c = _dot(h, W1b, ((1,), (1,))) + b1_ref[...]

    seg_t = _dot(sums, W1a, ((1,), (1,)))
    sum_y = (jnp.sum(seg_t, axis=0, keepdims=True)
             + jnp.sum(c * cnt, axis=0, keepdims=True))
    G = _dot(x, x, ((0,), (0,)))
    W1aG = _dot(W1a, G, ((1,), (0,)))
    ones_row = jnp.ones((1, d), dtype=jnp.float32)
    sumsq_t = _dot(ones_row, W1a * W1aG, ((1,), (1,)))
    sumsq_y = (sumsq_t
               + 2.0 * jnp.sum(c * seg_t, axis=0, keepdims=True)
               + jnp.sum(c * c * cnt, axis=0, keepdims=True))
    mean = sum_y / nf
    var = sumsq_y / nf - mean * mean
    a = gamma_ref[...] * lax.rsqrt(var + _EPS)
    bsh = beta_ref[...] - mean * a

    riota = lax.broadcasted_iota(jnp.int32, (d, d), 0)
    ciota = lax.broadcasted_iota(jnp.int32, (d, d), 1)
    eyef = (riota == ciota).astype(jnp.float32)
    a_col = _dot(eyef, a, ((1,), (1,)))
    W1a_s = W1a * a_col
    c2 = c * a + bsh

    t2 = _dot(x, W1a_s, ((1,), (1,)))
    s2 = _dot(maskT, c2, ((0,), (0,)))
    out_ref[...] = jnp.maximum(t2 + s2, 0.0)


def kernel(p, x, o, W1, b1, gamma, beta, W2, b2):
    del p
    n, d = x.shape
    nb = o.shape[0]
    o2 = o.reshape(nb, 1)
    return pl.pallas_call(
        _body,
        out_shape=jax.ShapeDtypeStruct((n, d), x.dtype),
    )(o2, x, W1, b1.reshape(1, d), gamma.reshape(1, d), beta.reshape(1, d),
      W2, b2.reshape(1, d))


# speedup vs baseline: 19.4300x; 3.1818x over previous
import jax
import jax.numpy as jnp
from jax import lax
from jax.experimental import pallas as pl

_EPS = 1e-5


def _dot(a, b, dims):
    return lax.dot_general(a, b, (dims, ((), ())),
                           preferred_element_type=jnp.float32)


def _body(o_ref, xT_ref, W1_ref, b1_ref, gamma_ref, beta_ref, W2_ref, b2_ref,
          outT_ref):
    d, n = xT_ref.shape
    nb = o_ref.shape[1]
    nf = jnp.float32(n)
    xT = xT_ref[...]
    of_row = o_ref[...].astype(jnp.float32)

    eye_b = (lax.broadcasted_iota(jnp.int32, (nb, nb), 0)
             == lax.broadcasted_iota(jnp.int32, (nb, nb), 1)
             ).astype(jnp.float32)
    eye_d = (lax.broadcasted_iota(jnp.int32, (d, d), 0)
             == lax.broadcasted_iota(jnp.int32, (d, d), 1)
             ).astype(jnp.float32)

    o_col = _dot(eye_b, of_row, ((1,), (1,)))
    prev_col = jnp.concatenate(
        [jnp.zeros((1, 1), jnp.float32), o_col[:-1, :]], axis=0)
    cnt_col = o_col - prev_col
    cnt_row = of_row - jnp.concatenate(
        [jnp.zeros((1, 1), jnp.float32), of_row[:, :-1]], axis=1)

    vecs = jnp.concatenate(
        [b1_ref[...], gamma_ref[...], beta_ref[...], b2_ref[...]], axis=0)
    vecs_col = _dot(eye_d, vecs, ((1,), (1,)))
    b1_col = vecs_col[:, 0:1]
    gamma_col = vecs_col[:, 1:2]
    beta_col = vecs_col[:, 2:3]
    b2_col = vecs_col[:, 3:4]

    rf = lax.broadcasted_iota(jnp.int32, (nb, n), 1).astype(jnp.float32)
    ltf = (rf < o_col).astype(jnp.float32)
    lt_prev = jnp.concatenate(
        [jnp.zeros((1, n), jnp.float32), ltf[:-1, :]], axis=0)
    maskT = ltf - lt_prev

    sums_T = _dot(xT, maskT, ((1,), (1,)))
    m_T = sums_T * (1.0 / cnt_row)
    h_T = jnp.maximum(_dot(W2_ref[...], m_T, ((1,), (0,))) + b2_col, 0.0)
    W1 = W1_ref[...]
    W1a = W1[:, :d]
    W1b = W1[:, d:]
    c_T = _dot(W1b, h_T, ((1,), (0,))) + b1_col

    seg_t_T = _dot(W1a, sums_T, ((1,), (0,)))
    sum_y = (jnp.sum(seg_t_T, axis=1, keepdims=True)
             + jnp.sum(c_T * cnt_row, axis=1, keepdims=True))
    G = _dot(xT, xT, ((1,), (1,)))
    W1aG = _dot(W1a, G, ((1,), (0,)))
    sumsq_t = jnp.sum(W1a * W1aG, axis=1, keepdims=True)
    sumsq_y = (sumsq_t
               + 2.0 * jnp.sum(c_T * seg_t_T, axis=1, keepdims=True)
               + jnp.sum(c_T * c_T * cnt_row, axis=1, keepdims=True))
    mean = sum_y / nf
    var = sumsq_y / nf - mean * mean
    a_col = gamma_col * lax.rsqrt(var + _EPS)
    bsh_col = beta_col - mean * a_col

    W1a_s = W1a * a_col
    c2_T = c_T * a_col + bsh_col

    t2 = _dot(W1a_s, xT, ((1,), (0,)))
    s2 = _dot(c2_T, maskT, ((1,), (0,)))
    outT_ref[...] = jnp.maximum(t2 + s2, 0.0)


def kernel(p, x, o, W1, b1, gamma, beta, W2, b2):
    del p
    n, d = x.shape
    nb = o.shape[0]
    xT = jnp.swapaxes(x, 0, 1)
    outT = pl.pallas_call(
        _body,
        out_shape=jax.ShapeDtypeStruct((d, n), x.dtype),
    )(o.reshape(1, nb), xT, W1, b1.reshape(1, d), gamma.reshape(1, d),
      beta.reshape(1, d), W2, b2.reshape(1, d))
    return jnp.swapaxes(outT, 0, 1)
